# Initial kernel scaffold; baseline (speedup 1.0000x reference)
import jax
import jax.numpy as jnp
from jax.experimental import pallas as pl


def kernel(x, edge_index, edge_type, pool_indices, W_rel, W_root, bias):
    raise NotImplementedError("write your pallas kernel here")


# speedup vs baseline: 54.0918x; 54.0918x over previous
import functools

import jax
import jax.numpy as jnp
from jax import lax
from jax.experimental import pallas as pl
from jax.experimental.pallas import tpu as pltpu
from jax.experimental.pallas import tpu_sc as plsc

NC = 2
NS = 16
L = 16


def _sc_kernel(N, E, D, R, P):
    NW = NC * NS
    EW = E // NW
    EC = 2000
    G = 128
    ROWS = R * P + NS * 8
    STRIPE = ROWS // NS
    DUMMY = R * P
    CAP = EC + G
    PPT = P // NS
    DEGW = 128
    DEGR = 80

    mesh = plsc.VectorSubcoreMesh(core_axis_name="c", subcore_axis_name="s")

    @functools.partial(
        pl.kernel,
        mesh=mesh,
        compiler_params=pltpu.CompilerParams(needs_layout_passes=False),
        out_type=[
            jax.ShapeDtypeStruct((NC, ROWS, D), jnp.float32),
            jax.ShapeDtypeStruct((NC, DEGR, DEGW), jnp.float32),
            jax.ShapeDtypeStruct((P, D), jnp.float32),
            jax.ShapeDtypeStruct((P,), jnp.int32),
        ],
        scratch_types=[
            pltpu.VMEM((N,), jnp.int32),
            pltpu.VMEM((P,), jnp.int32),
            pltpu.VMEM((EC,), jnp.int32),
            pltpu.VMEM((EC,), jnp.int32),
            pltpu.VMEM((EC,), jnp.int32),
            pltpu.VMEM((CAP,), jnp.int32),
            pltpu.VMEM((CAP,), jnp.int32),
            pltpu.VMEM((DEGR, DEGW), jnp.float32),
            pltpu.VMEM((DEGR,), jnp.int32),
            pltpu.VMEM((G,), jnp.int32),
            pltpu.VMEM((G,), jnp.int32),
            pltpu.VMEM((PPT,), jnp.int32),
            pltpu.VMEM((G, D), jnp.float32),
            pltpu.VMEM((PPT, D), jnp.float32),
            pltpu.VMEM_SHARED((ROWS, D), jnp.float32),
            pltpu.VMEM_SHARED((DEGR, DEGW), jnp.float32),
            pltpu.SemaphoreType.DMA,
            pltpu.SemaphoreType.DMA,
        ],
    )
    def sc(x_hbm, src_hbm, dst_hbm, et_hbm, pool_hbm,
           pre_out, deg_out, xpool_out, rep_out,
           slot_v, pool_v, src_c, dst_c, et_c, flt_src, flt_idx, deg_v,
           drows, sidx, didx, pidx, rows_v, xrows, pre_sp, deg_sp, sem, sem2):
        c = lax.axis_index("c")
        s = lax.axis_index("s")
        w = c * NS + s
        ebase = w * EW

        pltpu.sync_copy(pool_hbm, pool_v)

        neg1 = jnp.full((L,), -1, jnp.int32)

        def init_slot(i, _):
            slot_v[pl.ds(i * L, L)] = neg1
            return 0
        lax.fori_loop(0, N // L, init_slot, 0)
        if N % L:
            slot_v[pl.ds(N - L, L)] = neg1

        lane = lax.iota(jnp.int32, L)
        lane_masks = [lane == j for j in range(L)]

        def set_slot(i, _):
            pv = pool_v[pl.ds(i * L, L)]
            pvals = jnp.full((L,), i * L, jnp.int32) + lane
            for j in range(L):
                plsc.store_scatter(slot_v, [pv], pvals, mask=lane_masks[j])
            return 0
        lax.fori_loop(0, P // L, set_slot, 0)

        zf = jnp.zeros((L,), jnp.float32)

        def zero_deg(i, _):
            for j in range(DEGW // L):
                deg_v[i, pl.ds(j * L, L)] = zf
            return 0
        lax.fori_loop(0, DEGR, zero_deg, 0)

        def fill_drows(i, _):
            drows[pl.ds(i * L, L)] = jnp.full((L,), i * L, jnp.int32) + lane
            return 0
        lax.fori_loop(0, DEGR // L, fill_drows, 0)

        def zero_rows(i, _):
            for j in range(D // L):
                rows_v[i, pl.ds(j * L, L)] = zf
            return 0
        lax.fori_loop(0, G, zero_rows, 0)

        rbase = s * STRIPE
        nfull = STRIPE // G
        for k in range(nfull):
            pltpu.sync_copy(rows_v, pre_sp.at[pl.ds(rbase + k * G, G)])
        rem = STRIPE - nfull * G
        if rem:
            pltpu.sync_copy(rows_v.at[pl.ds(0, rem)],
                            pre_sp.at[pl.ds(rbase + nfull * G, rem)])

        @pl.when(s == 0)
        def _():
            pltpu.sync_copy(deg_v, deg_sp)

        plsc.subcore_barrier()

        ones_f = jnp.ones((L,), jnp.float32)

        def gs_body(g, _):
            base = g * G
            for j in range(G // L):
                sidx[pl.ds(j * L, L)] = flt_src[pl.ds(base + j * L, L)]
                didx[pl.ds(j * L, L)] = flt_idx[pl.ds(base + j * L, L)]
            pltpu.async_copy(x_hbm.at[sidx], rows_v, sem).wait()
            pltpu.sync_copy(rows_v, pre_sp.at[didx], add=True)
            return 0

        def chunk_body(k, ptr):
            off = ebase + k * EC
            pltpu.sync_copy(src_hbm.at[pl.ds(off, EC)], src_c)
            pltpu.sync_copy(dst_hbm.at[pl.ds(off, EC)], dst_c)
            pltpu.sync_copy(et_hbm.at[pl.ds(off, EC)], et_c)

            def vec_body(i, ptr):
                dv = dst_c[pl.ds(i * L, L)]
                sv = src_c[pl.ds(i * L, L)]
                tv = et_c[pl.ds(i * L, L)]
                sl = plsc.load_gather(slot_v, [dv])
                msk = sl >= 0
                bucket = tv * P + jnp.maximum(sl, 0)
                dcnt, dlast = plsc.scan_count(bucket, mask=msk)
                plsc.addupdate_scatter(
                    deg_v,
                    [lax.shift_right_logical(bucket, 7),
                     lax.bitwise_and(bucket, jnp.int32(DEGW - 1))],
                    dcnt.astype(jnp.float32), mask=dlast)
                plsc.store_compressed(flt_src.at[pl.ds(ptr, L)], sv, mask=msk)
                plsc.store_compressed(flt_idx.at[pl.ds(ptr, L)], bucket, mask=msk)
                return ptr + jnp.sum(msk.astype(jnp.int32))
            ptr = lax.fori_loop(0, EC // L, vec_body, ptr)

            nav = ptr // G
            lax.fori_loop(0, nav, gs_body, 0)
            rem_base = nav * G
            for j in range(G // L):
                sv = flt_src[pl.ds(rem_base + j * L, L)]
                bv = flt_idx[pl.ds(rem_base + j * L, L)]
                flt_src[pl.ds(j * L, L)] = sv
                flt_idx[pl.ds(j * L, L)] = bv
            return ptr - rem_base

        ptr = lax.fori_loop(0, EW // EC, chunk_body, jnp.int32(0))

        zi = jnp.zeros((L,), jnp.int32)
        dmy = jnp.full((L,), DUMMY, jnp.int32)
        for j in range(G // L):
            flt_src[pl.ds(ptr + j * L, L)] = zi
            flt_idx[pl.ds(ptr + j * L, L)] = dmy
        lax.fori_loop(0, (ptr + G - 1) // G, gs_body, 0)

        pltpu.sync_copy(deg_v, deg_sp.at[drows], add=True)

        @pl.when(c == 0)
        def _():
            pbase = s * PPT
            for j in range(PPT // L):
                pidx[pl.ds(j * L, L)] = pool_v[pl.ds(pbase + j * L, L)]
            pltpu.async_copy(x_hbm.at[pidx], xrows, sem2).wait()
            pltpu.sync_copy(xrows, xpool_out.at[pl.ds(pbase, PPT)])
            for j in range(PPT // L):
                pv = pool_v[pl.ds(pbase + j * L, L)]
                pidx[pl.ds(j * L, L)] = plsc.load_gather(slot_v, [pv])
            pltpu.sync_copy(pidx, rep_out.at[pl.ds(pbase, PPT)])

        plsc.subcore_barrier()

        pltpu.sync_copy(pre_sp.at[pl.ds(rbase, STRIPE)],
                        pre_out.at[c, pl.ds(rbase, STRIPE)])

        @pl.when(s == 0)
        def _():
            pltpu.sync_copy(deg_sp, deg_out.at[c])

    return sc


def _tc_body(R, P, D, pre_ref, deg_ref, xp_ref, rep_ref, wrel_ref,
             wroot_ref, bias_ref, out_ref):
    RP = R * P
    pre = pre_ref[0, :RP, :] + pre_ref[1, :RP, :]
    deg = deg_ref[0, :RP, :] + deg_ref[1, :RP, :]
    norm = 1.0 / jnp.maximum(deg, 1.0)
    M = (pre * norm).reshape(R, P, D)
    hi = lax.Precision.HIGHEST
    acc = jnp.dot(M[0], wrel_ref[0], precision=hi,
                  preferred_element_type=jnp.float32)
    for r in range(1, R):
        acc += jnp.dot(M[r], wrel_ref[r], precision=hi,
                       preferred_element_type=jnp.float32)
    xp = xp_ref[...]
    root = jnp.dot(xp, wroot_ref[...], precision=hi,
                   preferred_element_type=jnp.float32)
    h = jnp.maximum(acc + root + bias_ref[...], 0.0)
    xb = xp[:, 0:3].astype(jnp.bfloat16).astype(jnp.float32)
    w = 4.0 * xb[:, 0:1] + xb[:, 1:2] + 2.0 * xb[:, 2:3]
    iota = lax.broadcasted_iota(jnp.int32, (P, P), 1)
    S = (rep_ref[...] == iota).astype(jnp.float32)
    wsum = jnp.sum(S * w, axis=0, keepdims=True)
    sw = jnp.sum(w) + 1e-9
    out_ref[...] = jnp.dot(wsum, h, precision=lax.Precision.HIGHEST,
                           preferred_element_type=jnp.float32) / sw


def kernel(x, edge_index, edge_type, pool_indices, W_rel, W_root, bias):
    N, D = x.shape
    E = edge_index.shape[1]
    R = W_rel.shape[0]
    P = pool_indices.shape[0]

    src = edge_index[0]
    dst = edge_index[1]

    sc = _sc_kernel(N, E, D, R, P)
    pre, deg, xpool, rep = sc(x, src, dst, edge_type, pool_indices)

    tc = pl.pallas_call(
        functools.partial(_tc_body, R, P, D),
        out_shape=jax.ShapeDtypeStruct((1, D), jnp.float32),
    )
    return tc(pre, deg.reshape(NC, -1, 1), xpool, rep.reshape(P, 1),
              W_rel, W_root, bias.reshape(1, D))


# speedup vs baseline: 56.1568x; 1.0382x over previous
import functools

import jax
import jax.numpy as jnp
from jax import lax
from jax.experimental import pallas as pl
from jax.experimental.pallas import tpu as pltpu
from jax.experimental.pallas import tpu_sc as plsc

NC = 2
NS = 16
L = 16


def _sc_kernel(N, E, D, R, P):
    NW = NC * NS
    EW = E // NW
    EC = 2000
    G = 128
    ROWS = R * P + NS * 8
    STRIPE = ROWS // NS
    DUMMY = R * P
    CAP = EC + G
    PPT = P // NS
    DEGW = 128
    DEGR = 80

    mesh = plsc.VectorSubcoreMesh(core_axis_name="c", subcore_axis_name="s")

    @functools.partial(
        pl.kernel,
        mesh=mesh,
        compiler_params=pltpu.CompilerParams(needs_layout_passes=False),
        out_type=[
            jax.ShapeDtypeStruct((NC, ROWS, D), jnp.float32),
            jax.ShapeDtypeStruct((NC, DEGR, DEGW), jnp.float32),
            jax.ShapeDtypeStruct((P, D), jnp.float32),
            jax.ShapeDtypeStruct((P,), jnp.int32),
        ],
        scratch_types=[
            pltpu.VMEM((N,), jnp.int32),
            pltpu.VMEM((P,), jnp.int32),
            pltpu.VMEM((EC,), jnp.int32),
            pltpu.VMEM((EC,), jnp.int32),
            pltpu.VMEM((EC,), jnp.int32),
            pltpu.VMEM((EC,), jnp.int32),
            pltpu.VMEM((EC,), jnp.int32),
            pltpu.VMEM((EC,), jnp.int32),
            pltpu.VMEM((CAP,), jnp.int32),
            pltpu.VMEM((CAP,), jnp.int32),
            pltpu.VMEM((DEGR, DEGW), jnp.float32),
            pltpu.VMEM((DEGR,), jnp.int32),
            pltpu.VMEM((G,), jnp.int32),
            pltpu.VMEM((G,), jnp.int32),
            pltpu.VMEM((PPT,), jnp.int32),
            pltpu.VMEM((G, D), jnp.float32),
            pltpu.VMEM_SHARED((ROWS, D), jnp.float32),
            pltpu.VMEM_SHARED((DEGR, DEGW), jnp.float32),
            pltpu.SemaphoreType.DMA,
            pltpu.SemaphoreType.DMA,
            pltpu.SemaphoreType.DMA,
        ],
    )
    def sc(x_hbm, src_hbm, dst_hbm, et_hbm, pool_hbm,
           pre_out, deg_out, xpool_out, rep_out,
           slot_v, pool_v, src_c, dst_c, et_c, src_c2, dst_c2, et_c2,
           flt_src, flt_idx, deg_v,
           drows, sidx, didx, pidx, rows_v, pre_sp, deg_sp,
           sem, sem2, sem3):
        c = lax.axis_index("c")
        s = lax.axis_index("s")
        w = c * NS + s
        ebase = w * EW

        pltpu.sync_copy(pool_hbm, pool_v)

        neg1 = jnp.full((L,), -1, jnp.int32)

        def init_slot(i, _):
            slot_v[pl.ds(i * L, L)] = neg1
            return 0
        lax.fori_loop(0, N // L, init_slot, 0)
        if N % L:
            slot_v[pl.ds(N - L, L)] = neg1

        lane = lax.iota(jnp.int32, L)

        def set_slot(i, _):
            pv = pool_v[pl.ds(i * L, L)]
            pvals = jnp.full((L,), i * L, jnp.int32) + lane
            _, plast = plsc.scan_count(pv)
            plsc.store_scatter(slot_v, [pv], pvals, mask=plast)
            return 0
        lax.fori_loop(0, P // L, set_slot, 0)

        zf = jnp.zeros((L,), jnp.float32)

        def zero_deg(i, _):
            for j in range(DEGW // L):
                deg_v[i, pl.ds(j * L, L)] = zf
            return 0
        lax.fori_loop(0, DEGR, zero_deg, 0)

        def fill_drows(i, _):
            drows[pl.ds(i * L, L)] = jnp.full((L,), i * L, jnp.int32) + lane
            return 0
        lax.fori_loop(0, DEGR // L, fill_drows, 0)

        def zero_rows(i, _):
            for j in range(D // L):
                rows_v[i, pl.ds(j * L, L)] = zf
            return 0
        lax.fori_loop(0, G, zero_rows, 0)

        rbase = s * STRIPE
        nfull = STRIPE // G
        for k in range(nfull):
            pltpu.sync_copy(rows_v, pre_sp.at[pl.ds(rbase + k * G, G)])
        rem = STRIPE - nfull * G
        if rem:
            pltpu.sync_copy(rows_v.at[pl.ds(0, rem)],
                            pre_sp.at[pl.ds(rbase + nfull * G, rem)])

        @pl.when(s == 0)
        def _():
            pltpu.sync_copy(deg_v, deg_sp)

        plsc.subcore_barrier()

        ones_f = jnp.ones((L,), jnp.float32)

        def gs_body(g, _):
            base = g * G
            for j in range(G // L):
                sidx[pl.ds(j * L, L)] = flt_src[pl.ds(base + j * L, L)]
                didx[pl.ds(j * L, L)] = flt_idx[pl.ds(base + j * L, L)]
            pltpu.async_copy(x_hbm.at[sidx], rows_v, sem).wait()
            pltpu.sync_copy(rows_v, pre_sp.at[didx], add=True)
            return 0

        sbufs = [(src_c, dst_c, et_c), (src_c2, dst_c2, et_c2)]

        def stage(k, b):
            off = ebase + k * EC
            return [
                pltpu.async_copy(src_hbm.at[pl.ds(off, EC)], sbufs[b][0], sem3),
                pltpu.async_copy(dst_hbm.at[pl.ds(off, EC)], sbufs[b][1], sem3),
                pltpu.async_copy(et_hbm.at[pl.ds(off, EC)], sbufs[b][2], sem3),
            ]

        NCHUNK = EW // EC
        descs = stage(0, 0)
        ptr = jnp.int32(0)
        for k in range(NCHUNK):
            for dsc in descs:
                dsc.wait()
            if k + 1 < NCHUNK:
                descs = stage(k + 1, (k + 1) % 2)
            sb, db, tb = sbufs[k % 2]

            def vec_body(i, ptr, sb=sb, db=db, tb=tb):
                dv = db[pl.ds(i * L, L)]
                sv = sb[pl.ds(i * L, L)]
                tv = tb[pl.ds(i * L, L)]
                sl = plsc.load_gather(slot_v, [dv])
                msk = sl >= 0
                bucket = tv * P + jnp.maximum(sl, 0)
                dcnt, dlast = plsc.scan_count(bucket, mask=msk)
                plsc.addupdate_scatter(
                    deg_v,
                    [lax.shift_right_logical(bucket, 7),
                     lax.bitwise_and(bucket, jnp.int32(DEGW - 1))],
                    dcnt.astype(jnp.float32), mask=dlast)
                plsc.store_compressed(flt_src.at[pl.ds(ptr, L)], sv, mask=msk)
                plsc.store_compressed(flt_idx.at[pl.ds(ptr, L)], bucket, mask=msk)
                return ptr + jnp.sum(msk.astype(jnp.int32))
            ptr = lax.fori_loop(0, EC // L, vec_body, ptr)

            nav = ptr // G
            lax.fori_loop(0, nav, gs_body, 0)
            rem_base = nav * G
            for j in range(G // L):
                sv = flt_src[pl.ds(rem_base + j * L, L)]
                bv = flt_idx[pl.ds(rem_base + j * L, L)]
                flt_src[pl.ds(j * L, L)] = sv
                flt_idx[pl.ds(j * L, L)] = bv
            ptr = ptr - rem_base

        zi = jnp.zeros((L,), jnp.int32)
        dmy = jnp.full((L,), DUMMY, jnp.int32)
        for j in range(G // L):
            flt_src[pl.ds(ptr + j * L, L)] = zi
            flt_idx[pl.ds(ptr + j * L, L)] = dmy
        lax.fori_loop(0, (ptr + G - 1) // G, gs_body, 0)

        pltpu.sync_copy(deg_v, deg_sp.at[drows], add=True)

        @pl.when(c == 0)
        def _():
            pbase = s * PPT
            for j in range(PPT // L):
                pidx[pl.ds(j * L, L)] = pool_v[pl.ds(pbase + j * L, L)]
            pltpu.async_copy(x_hbm.at[pidx], rows_v.at[pl.ds(0, PPT)],
                             sem2).wait()
            pltpu.sync_copy(rows_v.at[pl.ds(0, PPT)],
                            xpool_out.at[pl.ds(pbase, PPT)])
            for j in range(PPT // L):
                pv = pool_v[pl.ds(pbase + j * L, L)]
                pidx[pl.ds(j * L, L)] = plsc.load_gather(slot_v, [pv])
            pltpu.sync_copy(pidx, rep_out.at[pl.ds(pbase, PPT)])

        plsc.subcore_barrier()

        pltpu.sync_copy(pre_sp.at[pl.ds(rbase, STRIPE)],
                        pre_out.at[c, pl.ds(rbase, STRIPE)])

        @pl.when(s == 0)
        def _():
            pltpu.sync_copy(deg_sp, deg_out.at[c])

    return sc


def _tc_body(R, P, D, pre_ref, deg_ref, xp_ref, rep_ref, wrel_ref,
             wroot_ref, bias_ref, out_ref):
    RP = R * P
    pre = pre_ref[0, :RP, :] + pre_ref[1, :RP, :]
    deg = deg_ref[0, :RP, :] + deg_ref[1, :RP, :]
    norm = 1.0 / jnp.maximum(deg, 1.0)
    M = (pre * norm).reshape(R, P, D)
    hi = lax.Precision.HIGHEST
    acc = jnp.dot(M[0], wrel_ref[0], precision=hi,
                  preferred_element_type=jnp.float32)
    for r in range(1, R):
        acc += jnp.dot(M[r], wrel_ref[r], precision=hi,
                       preferred_element_type=jnp.float32)
    xp = xp_ref[...]
    root = jnp.dot(xp, wroot_ref[...], precision=hi,
                   preferred_element_type=jnp.float32)
    h = jnp.maximum(acc + root + bias_ref[...], 0.0)
    xb = xp[:, 0:3].astype(jnp.bfloat16).astype(jnp.float32)
    w = 4.0 * xb[:, 0:1] + xb[:, 1:2] + 2.0 * xb[:, 2:3]
    iota = lax.broadcasted_iota(jnp.int32, (P, P), 1)
    S = (rep_ref[...] == iota).astype(jnp.float32)
    wsum = jnp.sum(S * w, axis=0, keepdims=True)
    sw = jnp.sum(w) + 1e-9
    out_ref[...] = jnp.dot(wsum, h, precision=lax.Precision.HIGHEST,
                           preferred_element_type=jnp.float32) / sw


def kernel(x, edge_index, edge_type, pool_indices, W_rel, W_root, bias):
    N, D = x.shape
    E = edge_index.shape[1]
    R = W_rel.shape[0]
    P = pool_indices.shape[0]

    src = edge_index[0]
    dst = edge_index[1]

    sc = _sc_kernel(N, E, D, R, P)
    pre, deg, xpool, rep = sc(x, src, dst, edge_type, pool_indices)

    tc = pl.pallas_call(
        functools.partial(_tc_body, R, P, D),
        out_shape=jax.ShapeDtypeStruct((1, D), jnp.float32),
    )
    return tc(pre, deg.reshape(NC, -1, 1), xpool, rep.reshape(P, 1),
              W_rel, W_root, bias.reshape(1, D))


# speedup vs baseline: 65.3054x; 1.1629x over previous
import functools

import jax
import jax.numpy as jnp
from jax import lax
from jax.experimental import pallas as pl
from jax.experimental.pallas import tpu as pltpu
from jax.experimental.pallas import tpu_sc as plsc

NC = 2
NS = 16
L = 16


def _sc_kernel(N, E, D, R, P):
    NW = NC * NS
    EW = E // NW
    EC = 2000
    G = 64
    ROWS = R * P + NS * 8
    STRIPE = ROWS // NS
    DUMMY = R * P
    CAP = EC + G
    PPT = P // NS
    DEGW = 128
    DEGR = 80

    mesh = plsc.VectorSubcoreMesh(core_axis_name="c", subcore_axis_name="s")

    @functools.partial(
        pl.kernel,
        mesh=mesh,
        compiler_params=pltpu.CompilerParams(needs_layout_passes=False),
        out_type=[
            jax.ShapeDtypeStruct((NC, ROWS, D), jnp.float32),
            jax.ShapeDtypeStruct((NC, DEGR, DEGW), jnp.float32),
            jax.ShapeDtypeStruct((P, D), jnp.float32),
            jax.ShapeDtypeStruct((P,), jnp.int32),
        ],
        scratch_types=[
            pltpu.VMEM((N,), jnp.int32),
            pltpu.VMEM((P,), jnp.int32),
            pltpu.VMEM((EC,), jnp.int32),
            pltpu.VMEM((EC,), jnp.int32),
            pltpu.VMEM((EC,), jnp.int32),
            pltpu.VMEM((EC,), jnp.int32),
            pltpu.VMEM((EC,), jnp.int32),
            pltpu.VMEM((EC,), jnp.int32),
            pltpu.VMEM((CAP,), jnp.int32),
            pltpu.VMEM((CAP,), jnp.int32),
            pltpu.VMEM((DEGR, DEGW), jnp.float32),
            pltpu.VMEM((DEGR,), jnp.int32),
            pltpu.VMEM((2, G), jnp.int32),
            pltpu.VMEM((2, G), jnp.int32),
            pltpu.VMEM((PPT,), jnp.int32),
            pltpu.VMEM((2, G, D), jnp.float32),
            pltpu.VMEM_SHARED((ROWS, D), jnp.float32),
            pltpu.VMEM_SHARED((DEGR, DEGW), jnp.float32),
            pltpu.SemaphoreType.DMA,
            pltpu.SemaphoreType.DMA,
            pltpu.SemaphoreType.DMA,
        ],
    )
    def sc(x_hbm, src_hbm, dst_hbm, et_hbm, pool_hbm,
           pre_out, deg_out, xpool_out, rep_out,
           slot_v, pool_v, src_c, dst_c, et_c, src_c2, dst_c2, et_c2,
           flt_src, flt_idx, deg_v,
           drows, sidx, didx, pidx, rows_v, pre_sp, deg_sp,
           sem, sem2, sem3):
        c = lax.axis_index("c")
        s = lax.axis_index("s")
        w = c * NS + s
        ebase = w * EW

        pltpu.sync_copy(pool_hbm, pool_v)

        neg1 = jnp.full((L,), -1, jnp.int32)

        def init_slot(i, _):
            slot_v[pl.ds(i * L, L)] = neg1
            return 0
        lax.fori_loop(0, N // L, init_slot, 0)
        if N % L:
            slot_v[pl.ds(N - L, L)] = neg1

        lane = lax.iota(jnp.int32, L)

        def set_slot(i, _):
            pv = pool_v[pl.ds(i * L, L)]
            pvals = jnp.full((L,), i * L, jnp.int32) + lane
            _, plast = plsc.scan_count(pv)
            plsc.store_scatter(slot_v, [pv], pvals, mask=plast)
            return 0
        lax.fori_loop(0, P // L, set_slot, 0)

        zf = jnp.zeros((L,), jnp.float32)

        def zero_deg(i, _):
            for j in range(DEGW // L):
                deg_v[i, pl.ds(j * L, L)] = zf
            return 0
        lax.fori_loop(0, DEGR, zero_deg, 0)

        def fill_drows(i, _):
            drows[pl.ds(i * L, L)] = jnp.full((L,), i * L, jnp.int32) + lane
            return 0
        lax.fori_loop(0, DEGR // L, fill_drows, 0)

        def zero_rows(i, _):
            for j in range(D // L):
                rows_v[0, i, pl.ds(j * L, L)] = zf
                rows_v[1, i, pl.ds(j * L, L)] = zf
            return 0
        lax.fori_loop(0, G, zero_rows, 0)

        rbase = s * STRIPE
        nfull = STRIPE // G
        for k in range(nfull):
            pltpu.sync_copy(rows_v.at[k % 2],
                            pre_sp.at[pl.ds(rbase + k * G, G)])
        rem = STRIPE - nfull * G
        if rem:
            pltpu.sync_copy(rows_v.at[0, pl.ds(0, rem)],
                            pre_sp.at[pl.ds(rbase + nfull * G, rem)])

        @pl.when(s == 0)
        def _():
            pltpu.sync_copy(deg_v, deg_sp)

        plsc.subcore_barrier()

        ones_f = jnp.ones((L,), jnp.float32)

        def fill_idx(g, h):
            base = g * G
            for j in range(G // L):
                sidx[h, pl.ds(j * L, L)] = flt_src[pl.ds(base + j * L, L)]
                didx[h, pl.ds(j * L, L)] = flt_idx[pl.ds(base + j * L, L)]

        def gs_pairs(nav):
            def gs_pair(i, _):
                g0 = 2 * i
                g1 = g0 + 1
                fill_idx(g0, 0)
                d0 = pltpu.async_copy(x_hbm.at[sidx.at[0]],
                                      rows_v.at[0], sem)

                @pl.when(g1 < nav)
                def _():
                    fill_idx(g1, 1)
                    pltpu.async_copy(x_hbm.at[sidx.at[1]],
                                     rows_v.at[1], sem2)

                d0.wait()
                pltpu.sync_copy(rows_v.at[0], pre_sp.at[didx.at[0]],
                                add=True)

                @pl.when(g1 < nav)
                def _():
                    pltpu.make_async_copy(x_hbm.at[sidx.at[1]],
                                          rows_v.at[1], sem2).wait()
                    pltpu.sync_copy(rows_v.at[1], pre_sp.at[didx.at[1]],
                                    add=True)
                return 0
            lax.fori_loop(0, (nav + 1) // 2, gs_pair, 0)

        sbufs = [(src_c, dst_c, et_c), (src_c2, dst_c2, et_c2)]

        def stage(k, b):
            off = ebase + k * EC
            return [
                pltpu.async_copy(src_hbm.at[pl.ds(off, EC)], sbufs[b][0], sem3),
                pltpu.async_copy(dst_hbm.at[pl.ds(off, EC)], sbufs[b][1], sem3),
                pltpu.async_copy(et_hbm.at[pl.ds(off, EC)], sbufs[b][2], sem3),
            ]

        NCHUNK = EW // EC
        descs = stage(0, 0)
        ptr = jnp.int32(0)
        for k in range(NCHUNK):
            for dsc in descs:
                dsc.wait()
            if k + 1 < NCHUNK:
                descs = stage(k + 1, (k + 1) % 2)
            sb, db, tb = sbufs[k % 2]

            def vec_body(i, ptr, sb=sb, db=db, tb=tb):
                dv = db[pl.ds(i * L, L)]
                sv = sb[pl.ds(i * L, L)]
                tv = tb[pl.ds(i * L, L)]
                sl = plsc.load_gather(slot_v, [dv])
                msk = sl >= 0
                bucket = tv * P + jnp.maximum(sl, 0)
                dcnt, dlast = plsc.scan_count(bucket, mask=msk)
                plsc.addupdate_scatter(
                    deg_v,
                    [lax.shift_right_logical(bucket, 7),
                     lax.bitwise_and(bucket, jnp.int32(DEGW - 1))],
                    dcnt.astype(jnp.float32), mask=dlast)
                plsc.store_compressed(flt_src.at[pl.ds(ptr, L)], sv, mask=msk)
                plsc.store_compressed(flt_idx.at[pl.ds(ptr, L)], bucket, mask=msk)
                return ptr + plsc.all_reduce_population_count(msk)[0]
            ptr = lax.fori_loop(0, EC // L, vec_body, ptr)

            nav = ptr // G
            gs_pairs(nav)
            rem_base = nav * G
            for j in range(G // L):
                sv = flt_src[pl.ds(rem_base + j * L, L)]
                bv = flt_idx[pl.ds(rem_base + j * L, L)]
                flt_src[pl.ds(j * L, L)] = sv
                flt_idx[pl.ds(j * L, L)] = bv
            ptr = ptr - rem_base

        zi = jnp.zeros((L,), jnp.int32)
        dmy = jnp.full((L,), DUMMY, jnp.int32)
        for j in range(G // L):
            flt_src[pl.ds(ptr + j * L, L)] = zi
            flt_idx[pl.ds(ptr + j * L, L)] = dmy
        gs_pairs((ptr + G - 1) // G)

        pltpu.sync_copy(deg_v, deg_sp.at[drows], add=True)

        @pl.when(c == 0)
        def _():
            pbase = s * PPT
            for j in range(PPT // L):
                pidx[pl.ds(j * L, L)] = pool_v[pl.ds(pbase + j * L, L)]
            pltpu.async_copy(x_hbm.at[pidx], rows_v.at[0], sem2).wait()
            pltpu.sync_copy(rows_v.at[0], xpool_out.at[pl.ds(pbase, PPT)])
            for j in range(PPT // L):
                pv = pool_v[pl.ds(pbase + j * L, L)]
                pidx[pl.ds(j * L, L)] = plsc.load_gather(slot_v, [pv])
            pltpu.sync_copy(pidx, rep_out.at[pl.ds(pbase, PPT)])

        plsc.subcore_barrier()

        pltpu.sync_copy(pre_sp.at[pl.ds(rbase, STRIPE)],
                        pre_out.at[c, pl.ds(rbase, STRIPE)])

        @pl.when(s == 0)
        def _():
            pltpu.sync_copy(deg_sp, deg_out.at[c])

    return sc


def _tc_body(R, P, D, pre_ref, deg_ref, xp_ref, rep_ref, wrel_ref,
             wroot_ref, bias_ref, out_ref):
    RP = R * P
    pre = pre_ref[0, :RP, :] + pre_ref[1, :RP, :]
    deg = deg_ref[0, :RP, :] + deg_ref[1, :RP, :]
    norm = 1.0 / jnp.maximum(deg, 1.0)
    M = (pre * norm).reshape(R, P, D)
    hi = lax.Precision.HIGHEST
    acc = jnp.dot(M[0], wrel_ref[0], precision=hi,
                  preferred_element_type=jnp.float32)
    for r in range(1, R):
        acc += jnp.dot(M[r], wrel_ref[r], precision=hi,
                       preferred_element_type=jnp.float32)
    xp = xp_ref[...]
    root = jnp.dot(xp, wroot_ref[...], precision=hi,
                   preferred_element_type=jnp.float32)
    h = jnp.maximum(acc + root + bias_ref[...], 0.0)
    xb = xp[:, 0:3].astype(jnp.bfloat16).astype(jnp.float32)
    w = 4.0 * xb[:, 0:1] + xb[:, 1:2] + 2.0 * xb[:, 2:3]
    iota = lax.broadcasted_iota(jnp.int32, (P, P), 1)
    S = (rep_ref[...] == iota).astype(jnp.float32)
    wsum = jnp.sum(S * w, axis=0, keepdims=True)
    sw = jnp.sum(w) + 1e-9
    out_ref[...] = jnp.dot(wsum, h, precision=lax.Precision.HIGHEST,
                           preferred_element_type=jnp.float32) / sw


def kernel(x, edge_index, edge_type, pool_indices, W_rel, W_root, bias):
    N, D = x.shape
    E = edge_index.shape[1]
    R = W_rel.shape[0]
    P = pool_indices.shape[0]

    src = edge_index[0]
    dst = edge_index[1]

    sc = _sc_kernel(N, E, D, R, P)
    pre, deg, xpool, rep = sc(x, src, dst, edge_type, pool_indices)

    tc = pl.pallas_call(
        functools.partial(_tc_body, R, P, D),
        out_shape=jax.ShapeDtypeStruct((1, D), jnp.float32),
    )
    return tc(pre, deg.reshape(NC, -1, 1), xpool, rep.reshape(P, 1),
              W_rel, W_root, bias.reshape(1, D))
